# baseline (speedup 1.0000x reference)
import jax
import jax.numpy as jnp
from jax.experimental import pallas as pl
from jax.experimental.pallas import tpu as pltpu

_N = 64
_DIN = 128
_DH = 256
_PB = 128
_TRI = _N * (_N + 1) // 2
_G = (_TRI + _PB - 1) // _PB


def _pair_kernel(z_ref, adj_ref, W1a_ref, W1b_ref, W2_ref, outA_ref, outB_ref):
    step = pl.program_id(0)

    a = adj_ref[...]
    S = jnp.maximum(a, a.T)
    H0 = jnp.dot(z_ref[...], W1a_ref[...], preferred_element_type=jnp.float32)
    W12 = jnp.concatenate(
        [W1b_ref[...],
         jnp.concatenate([W1b_ref[1:2], W1b_ref[0:1], W1b_ref[2:]], axis=0)],
        axis=1)
    r = jax.lax.broadcasted_iota(jnp.int32, (_N, _N), 0)
    c = jax.lax.broadcasted_iota(jnp.int32, (_N, _N), 1)
    L = (c < r).astype(jnp.float32)
    U = (r < c).astype(jnp.float32)
    CS = jnp.dot(L, S, preferred_element_type=jnp.float32)
    RS = jnp.dot(S, U, preferred_element_type=jnp.float32)
    Sd = jnp.sum(S * (r == c).astype(jnp.float32), axis=0)
    PRE = jnp.concatenate([CS - Sd[None, :], RS], axis=1)

    W2v = W2_ref[...].reshape(1, 1, _DH)

    p3 = step * _PB + jax.lax.broadcasted_iota(jnp.int32, (_PB, 1, 1), 0)
    c2 = jax.lax.broadcasted_iota(jnp.int32, (_PB, _N), 1)
    p2 = p3[:, :, 0]
    u2 = jnp.sum((p2 >= (c2 * (c2 + 1)) // 2).astype(jnp.int32),
                 axis=1, keepdims=True) - 1
    l2 = p2 - (u2 * (u2 + 1)) // 2
    u3 = u2[:, :, None]
    l3 = l2[:, :, None]
    cu = c2 < u2
    ohu = (c2 == u2).astype(jnp.float32)
    ohl = (c2 == l2).astype(jnp.float32)
    cu_f = cu.astype(jnp.float32)
    cl_f = (c2 < l2).astype(jnp.float32)

    OH2 = jnp.concatenate([ohu, ohl], axis=0)
    SR = jnp.dot(OH2, S, preferred_element_type=jnp.float32)
    Su, Sl = SR[:_PB], SR[_PB:]
    Gp = jnp.dot(ohu, PRE, preferred_element_type=jnp.float32)
    CSu, RSu = Gp[:, :_N], Gp[:, _N:]

    RSul = jnp.sum(RSu * ohl, axis=1, keepdims=True)
    deg = 1.0 + jnp.where(cu, CSu + Su * cl_f, ohu * RSul)
    Dn = jax.lax.rsqrt(jnp.maximum(deg, 1.0))

    Pu = jnp.where(ohu > 0, 1.0, Su * cl_f)
    Pl = jnp.where(ohl > 0, 1.0, Sl * jnp.where(l2 < u2, cu_f, cl_f))
    Dnu = jnp.sum(ohu * Dn, axis=1, keepdims=True)
    Dnl = jnp.sum(ohl * Dn, axis=1, keepdims=True)
    cu_col = Dnu * Pu
    cl_col = Dnl * Pl
    t = Dn * (Dnu * Pu + Dnl * Pl)

    r3 = jax.lax.broadcasted_iota(jnp.int32, (_PB, _N, 1), 1)
    ru = (r3 < u3).astype(jnp.float32)
    re = (r3 == u3).astype(jnp.float32)
    rl = (r3 < l3).astype(jnp.float32)
    cuD = (cu_f * Dn)[:, None, :]
    clD = (cl_f * Dn)[:, None, :]
    ceD = (ohu * Dn)[:, None, :]
    rr = jax.lax.broadcasted_iota(jnp.int32, (_N, _N), 0)
    cc = jax.lax.broadcasted_iota(jnp.int32, (_N, _N), 1)
    C = jnp.where((rr == cc)[None, :, :],
                  Dn[:, None, :],
                  S[None, :, :] * (ru * cuD + re * clD + rl * ceD))

    le = jax.lax.broadcasted_iota(jnp.int32, (_PB, _N, 8), 2)
    E = (jnp.where(le == 0, cu_col[:, :, None], 0.0)
         + jnp.where(le == 1, cl_col[:, :, None], 0.0))
    Ef = E.reshape(_PB * _N, 8)
    G = jnp.dot(C.reshape(_PB * _N, _N), H0,
                preferred_element_type=jnp.float32)
    C12 = jnp.dot(Ef, W12, preferred_element_type=jnp.float32)
    C1, C2 = C12[:, :_DH], C12[:, _DH:]
    M1 = (G + C1).reshape(_PB, _N, _DH)
    M2 = (G + C2).reshape(_PB, _N, _DH)
    v1 = Dn * jnp.sum(jnp.maximum(M1, 0.0) * W2v, axis=2)
    v2 = Dn * jnp.sum(jnp.maximum(M2, 0.0) * W2v, axis=2)

    outA_ref[0, 0, :] = jnp.sum(t * v1, axis=1)
    outB_ref[0, 0, :] = jnp.sum(t * v2, axis=1)


def kernel(inputs, adj, W1, W2):
    W1a = W1[:_DIN]
    W1b = jnp.pad(W1[_DIN:], ((0, 6), (0, 0)))
    W2r = W2.reshape(1, _DH)
    outA, outB = pl.pallas_call(
        _pair_kernel,
        grid=(_G,),
        in_specs=[
            pl.BlockSpec((_N, _DIN), lambda s: (0, 0)),
            pl.BlockSpec((_N, _N), lambda s: (0, 0)),
            pl.BlockSpec((_DIN, _DH), lambda s: (0, 0)),
            pl.BlockSpec((8, _DH), lambda s: (0, 0)),
            pl.BlockSpec((1, _DH), lambda s: (0, 0)),
        ],
        out_specs=[pl.BlockSpec((1, 1, _PB), lambda s: (s, 0, 0)),
                   pl.BlockSpec((1, 1, _PB), lambda s: (s, 0, 0))],
        out_shape=[jax.ShapeDtypeStruct((_G, 1, _PB), jnp.float32),
                   jax.ShapeDtypeStruct((_G, 1, _PB), jnp.float32)],
        compiler_params=pltpu.CompilerParams(dimension_semantics=("parallel",)),
    )(inputs, adj, W1a, W1b, W2r)
    um = jnp.arange(_N)[:, None]
    lm = jnp.arange(_N)[None, :]
    tri = um * (um + 1) // 2 + lm
    A_sq = outA.reshape(-1)[tri]
    B_sq = outB.reshape(-1)[tri]
    return jnp.where(um >= lm, A_sq, B_sq.T).reshape(-1)


# speedup vs baseline: 1.0187x; 1.0187x over previous
import jax
import jax.numpy as jnp
from jax.experimental import pallas as pl
from jax.experimental.pallas import tpu as pltpu

_N = 64
_DIN = 128
_DH = 256
_PB = 128
_TRI = _N * (_N + 1) // 2
_G = (_TRI + _PB - 1) // _PB


def _pair_kernel(z_ref, adj_ref, W1a_ref, W1b_ref, W2_ref, outA_ref, outB_ref,
                 H0_ref, S_ref, PRE_ref, W12_ref):
    step = pl.program_id(0)

    @pl.when(step == 0)
    def _prologue():
        a = adj_ref[...]
        S = jnp.maximum(a, a.T)
        S_ref[...] = S
        H0_ref[...] = jnp.dot(z_ref[...], W1a_ref[...],
                              preferred_element_type=jnp.float32)
        W12_ref[:, :_DH] = W1b_ref[...]
        W12_ref[:, _DH:] = jnp.concatenate(
            [W1b_ref[1:2], W1b_ref[0:1], W1b_ref[2:]], axis=0)
        r = jax.lax.broadcasted_iota(jnp.int32, (_N, _N), 0)
        c = jax.lax.broadcasted_iota(jnp.int32, (_N, _N), 1)
        L = (c < r).astype(jnp.float32)
        U = (r < c).astype(jnp.float32)
        CS = jnp.dot(L, S, preferred_element_type=jnp.float32)
        RS = jnp.dot(S, U, preferred_element_type=jnp.float32)
        Sd = jnp.sum(S * (r == c).astype(jnp.float32), axis=0)
        PRE_ref[:, :_N] = CS - Sd[None, :]
        PRE_ref[:, _N:] = RS

    S = S_ref[...]
    W2v = W2_ref[...].reshape(1, 1, _DH)

    p3 = step * _PB + jax.lax.broadcasted_iota(jnp.int32, (_PB, 1, 1), 0)
    c2 = jax.lax.broadcasted_iota(jnp.int32, (_PB, _N), 1)
    p2 = p3[:, :, 0]
    u2 = jnp.sum((p2 >= (c2 * (c2 + 1)) // 2).astype(jnp.int32),
                 axis=1, keepdims=True) - 1
    l2 = p2 - (u2 * (u2 + 1)) // 2
    u3 = u2[:, :, None]
    l3 = l2[:, :, None]
    cu = c2 < u2
    ohu = (c2 == u2).astype(jnp.float32)
    ohl = (c2 == l2).astype(jnp.float32)
    cu_f = cu.astype(jnp.float32)
    cl_f = (c2 < l2).astype(jnp.float32)

    OH2 = jnp.concatenate([ohu, ohl], axis=0)
    SR = jnp.dot(OH2, S, preferred_element_type=jnp.float32)
    Su, Sl = SR[:_PB], SR[_PB:]
    Gp = jnp.dot(ohu, PRE_ref[...], preferred_element_type=jnp.float32)
    CSu, RSu = Gp[:, :_N], Gp[:, _N:]

    RSul = jnp.sum(RSu * ohl, axis=1, keepdims=True)
    deg = 1.0 + jnp.where(cu, CSu + Su * cl_f, ohu * RSul)
    Dn = jax.lax.rsqrt(jnp.maximum(deg, 1.0))

    Pu = jnp.where(ohu > 0, 1.0, Su * cl_f)
    Pl = jnp.where(ohl > 0, 1.0, Sl * jnp.where(l2 < u2, cu_f, cl_f))
    Dnu = jnp.sum(ohu * Dn, axis=1, keepdims=True)
    Dnl = jnp.sum(ohl * Dn, axis=1, keepdims=True)
    cu_col = Dnu * Pu
    cl_col = Dnl * Pl
    t = Dn * (Dnu * Pu + Dnl * Pl)

    r3 = jax.lax.broadcasted_iota(jnp.int32, (_PB, _N, 1), 1)
    ru = (r3 < u3).astype(jnp.float32)
    re = (r3 == u3).astype(jnp.float32)
    rl = (r3 < l3).astype(jnp.float32)
    cuD = (cu_f * Dn)[:, None, :]
    clD = (cl_f * Dn)[:, None, :]
    ceD = (ohu * Dn)[:, None, :]
    rr = jax.lax.broadcasted_iota(jnp.int32, (_N, _N), 0)
    cc = jax.lax.broadcasted_iota(jnp.int32, (_N, _N), 1)
    C = jnp.where((rr == cc)[None, :, :],
                  Dn[:, None, :],
                  S[None, :, :] * (ru * cuD + re * clD + rl * ceD))

    le = jax.lax.broadcasted_iota(jnp.int32, (_PB, _N, 8), 2)
    E = (jnp.where(le == 0, cu_col[:, :, None], 0.0)
         + jnp.where(le == 1, cl_col[:, :, None], 0.0))
    Ef = E.reshape(_PB * _N, 8)
    G = jnp.dot(C.reshape(_PB * _N, _N), H0_ref[...],
                preferred_element_type=jnp.float32)
    C12 = jnp.dot(Ef, W12_ref[...], preferred_element_type=jnp.float32)
    C1, C2 = C12[:, :_DH], C12[:, _DH:]
    M1 = (G + C1).reshape(_PB, _N, _DH)
    M2 = (G + C2).reshape(_PB, _N, _DH)
    v1 = Dn * jnp.sum(jnp.maximum(M1, 0.0) * W2v, axis=2)
    v2 = Dn * jnp.sum(jnp.maximum(M2, 0.0) * W2v, axis=2)

    outA_ref[0, 0, :] = jnp.sum(t * v1, axis=1)
    outB_ref[0, 0, :] = jnp.sum(t * v2, axis=1)


def kernel(inputs, adj, W1, W2):
    W1a = W1[:_DIN]
    W1b = jnp.pad(W1[_DIN:], ((0, 6), (0, 0)))
    W2r = W2.reshape(1, _DH)
    outA, outB = pl.pallas_call(
        _pair_kernel,
        grid=(_G,),
        in_specs=[
            pl.BlockSpec((_N, _DIN), lambda s: (0, 0)),
            pl.BlockSpec((_N, _N), lambda s: (0, 0)),
            pl.BlockSpec((_DIN, _DH), lambda s: (0, 0)),
            pl.BlockSpec((8, _DH), lambda s: (0, 0)),
            pl.BlockSpec((1, _DH), lambda s: (0, 0)),
        ],
        out_specs=[pl.BlockSpec((1, 1, _PB), lambda s: (s, 0, 0)),
                   pl.BlockSpec((1, 1, _PB), lambda s: (s, 0, 0))],
        out_shape=[jax.ShapeDtypeStruct((_G, 1, _PB), jnp.float32),
                   jax.ShapeDtypeStruct((_G, 1, _PB), jnp.float32)],
        scratch_shapes=[pltpu.VMEM((_N, _DH), jnp.float32),
                        pltpu.VMEM((_N, _N), jnp.float32),
                        pltpu.VMEM((_N, 2 * _N), jnp.float32),
                        pltpu.VMEM((8, 2 * _DH), jnp.float32)],
        compiler_params=pltpu.CompilerParams(dimension_semantics=("arbitrary",)),
    )(inputs, adj, W1a, W1b, W2r)
    um = jnp.arange(_N)[:, None]
    lm = jnp.arange(_N)[None, :]
    tri = um * (um + 1) // 2 + lm
    A_sq = outA.reshape(-1)[tri]
    B_sq = outB.reshape(-1)[tri]
    return jnp.where(um >= lm, A_sq, B_sq.T).reshape(-1)
